# baseline (speedup 1.0000x reference)
import functools

import jax
import jax.numpy as jnp
from jax import lax
from jax.experimental import pallas as pl
from jax.experimental.pallas import tpu as pltpu
from jax.experimental.pallas import tpu_sc as plsc

VOCAB = 1000000
EMB = 64
B = 4096
L = 200
CLS = 128

NC = 2
NS = 16
NW = NC * NS
ROWS_PER_W = B // NW
NLANE = EMB // 16

VROWS = VOCAB // 2
NCHUNK_FULL = VOCAB // 128
TAIL_V0 = NCHUNK_FULL * 128

NSLOT = 3


def _transpose_body(yT_hbm, tail_hbm, t2_hbm, in_v, out_v, tail_v, sems):
    wid = lax.axis_index("s") * NC + lax.axis_index("c")
    sem_in0, sem_in1, sem_out0, sem_out1 = sems
    sin = (sem_in0, sem_in1)
    sout = (sem_out0, sem_out1)

    lanes = lax.iota(jnp.int32, 16)
    R8 = jnp.right_shift(lanes, 1)
    C64 = jnp.left_shift(jnp.bitwise_and(lanes, 1), 6)

    def cid_of(k):
        return wid + NW * k

    def issue_in(k, slot):
        pltpu.async_copy(
            yT_hbm.at[:, pl.ds(128 * cid_of(k), 128)], in_v.at[slot],
            sin[slot])

    def wait_in(k, slot):
        pltpu.make_async_copy(
            yT_hbm.at[:, pl.ds(128 * cid_of(k), 128)], in_v.at[slot],
            sin[slot]).wait()

    def issue_out(k, slot):
        pltpu.async_copy(
            out_v.at[slot], t2_hbm.at[pl.ds(64 * cid_of(k), 64)],
            sout[slot])

    def wait_out(k, slot):
        pltpu.make_async_copy(
            out_v.at[slot], t2_hbm.at[pl.ds(64 * cid_of(k), 64)],
            sout[slot]).wait()

    def transpose_chunk(slot, src, n_groups):
        out2d = out_v.at[slot]

        def c_body(c, carry):
            colv = C64 + c
            for g in range(n_groups):
                vin = src[c, pl.ds(16 * g, 16)]
                plsc.store_scatter(out2d, [R8 + 8 * g, colv], vin)
            return carry

        lax.fori_loop(0, EMB, c_body, 0)

    NK = (NCHUNK_FULL + NW - 1) // NW

    @pl.when(cid_of(0) < NCHUNK_FULL)
    def _():
        issue_in(0, 0)

    @pl.when(cid_of(1) < NCHUNK_FULL)
    def _():
        issue_in(1, 1)

    def k_step(k, slot):
        @pl.when(cid_of(k) < NCHUNK_FULL)
        def _():
            wait_in(k, slot)

            @pl.when(k >= 2)
            def _():
                wait_out(k - 2, slot)

            transpose_chunk(slot, in_v.at[slot], 8)
            issue_out(k, slot)

            @pl.when(cid_of(k + 2) < NCHUNK_FULL)
            def _():
                issue_in(k + 2, slot)

    def pair_body(p, carry):
        k_step(2 * p, 0)
        k_step(2 * p + 1, 1)
        return carry

    lax.fori_loop(0, NK // 2, pair_body, 0)
    k_step(NK - 1, (NK - 1) % 2)

    for k in (NK - 2, NK - 1):
        @pl.when(cid_of(k) < NCHUNK_FULL)
        def _():
            wait_out(k, k % 2)

    @pl.when(wid == NW - 1)
    def _():
        pltpu.sync_copy(tail_hbm, tail_v)
        out2d = out_v.at[0]

        def tc_body(c, carry):
            colv = C64 + c
            for g in range(4):
                vin = tail_v[c, pl.ds(16 * g, 16)]
                plsc.store_scatter(out2d, [R8 + 8 * g, colv], vin)
            return carry

        lax.fori_loop(0, EMB, tc_body, 0)
        pltpu.sync_copy(out_v.at[0, pl.ds(0, 32)],
                        t2_hbm.at[pl.ds(TAIL_V0 // 2, 32)])


@functools.partial(
    pl.kernel,
    out_type=jax.ShapeDtypeStruct((VROWS, 2 * EMB), jnp.float32),
    mesh=plsc.VectorSubcoreMesh(core_axis_name="c", subcore_axis_name="s"),
    scratch_types=[
        pltpu.VMEM((2, EMB, 128), jnp.float32),
        pltpu.VMEM((2, EMB, 128), jnp.float32),
        pltpu.VMEM((EMB, EMB), jnp.float32),
    ] + [pltpu.SemaphoreType.DMA] * 4,
    compiler_params=pltpu.CompilerParams(
        use_tc_tiling_on_sc=True, needs_layout_passes=False),
)
def _transpose_sc(yT_hbm, tail_hbm, t2_hbm, in_v, out_v, tail_v, *sems):
    _transpose_body(yT_hbm, tail_hbm, t2_hbm, in_v, out_v, tail_v, sems)


def _pool_body(seq_hbm, table_hbm, out_hbm, idx_v, buf_v, out_v, sems):
    wid = lax.axis_index("s") * NC + lax.axis_index("c")
    base = wid * ROWS_PER_W
    pltpu.sync_copy(seq_hbm.at[pl.ds(base, ROWS_PER_W)], idx_v)

    GSTARTS = tuple(range(0, L - 16, 16)) + (L - 16,)

    def issue(row, slot):
        for st in GSTARTS:
            iv = jnp.right_shift(idx_v[row, pl.ds(st, 16)], 1)
            pltpu.async_copy(
                table_hbm.at[iv], buf_v.at[slot, pl.ds(st, 16)], sems[slot])

    def drain(row, slot):
        for st in GSTARTS:
            iv = jnp.right_shift(idx_v[row, pl.ds(st, 16)], 1)
            pltpu.make_async_copy(
                table_hbm.at[iv], buf_v.at[slot, pl.ds(st, 16)],
                sems[slot]).wait()

    IOTA = lax.iota(jnp.int32, 16)
    KVECS = [jnp.full((16,), k, jnp.int32) for k in range(16)]

    def dyn_bcast(v, kvec):
        return lax.gather(
            v, kvec[:, None],
            lax.GatherDimensionNumbers(
                offset_dims=(), collapsed_slice_dims=(0,),
                start_index_map=(0,)),
            (1,), mode=lax.GatherScatterMode.PROMISE_IN_BOUNDS)

    def compute(i, slot):
        buf2d = buf_v.at[slot]

        def group(acc, base16, krange):
            pv_off = jnp.left_shift(
                jnp.bitwise_and(idx_v[i, pl.ds(base16, 16)], 1), 6)
            new = list(acc)
            for k in krange:
                offv = dyn_bcast(pv_off, KVECS[k])
                tokv = jnp.full((16,), base16 + k, jnp.int32)
                for c in range(NLANE):
                    v = plsc.load_gather(buf2d, [tokv, offv + (IOTA + c * 16)])
                    new[c] = jnp.maximum(new[c], v)
            return tuple(new)

        def max_body(j, acc):
            return group(acc, 16 * j, range(16))

        init = tuple(jnp.full((16,), -jnp.inf, jnp.float32)
                     for _ in range(NLANE))
        acc = lax.fori_loop(0, L // 16, max_body, init)
        acc = group(acc, L - 16, range(8, 16))
        for c in range(NLANE):
            out_v[i, pl.ds(c * 16, 16)] = acc[c]

    for s in range(NSLOT):
        issue(s, s)

    def group_body(g, carry):
        r0 = NSLOT * g
        for s in range(NSLOT):
            drain(r0 + s, s)
            compute(r0 + s, s)

            @pl.when(r0 + s + NSLOT < ROWS_PER_W)
            def _():
                issue(r0 + s + NSLOT, s)
        return carry

    NFULL = ROWS_PER_W // NSLOT
    lax.fori_loop(0, NFULL, group_body, 0)
    for s in range(ROWS_PER_W - NSLOT * NFULL):
        drain(NSLOT * NFULL + s, s)
        compute(NSLOT * NFULL + s, s)

    pltpu.sync_copy(out_v, out_hbm.at[pl.ds(wid * ROWS_PER_W, ROWS_PER_W)])


@functools.partial(
    pl.kernel,
    out_type=jax.ShapeDtypeStruct((B, EMB), jnp.float32),
    mesh=plsc.VectorSubcoreMesh(core_axis_name="c", subcore_axis_name="s"),
    scratch_types=[
        pltpu.VMEM((ROWS_PER_W, L), jnp.int32),
        pltpu.VMEM((NSLOT, L, 2 * EMB), jnp.float32),
        pltpu.VMEM((ROWS_PER_W, EMB), jnp.float32),
    ] + [pltpu.SemaphoreType.DMA] * NSLOT,
    compiler_params=pltpu.CompilerParams(
        use_tc_tiling_on_sc=False, needs_layout_passes=False),
)
def _pool_sc(seq_hbm, table_hbm, out_hbm, idx_v, buf_v, out_v, *sems):
    _pool_body(seq_hbm, table_hbm, out_hbm, idx_v, buf_v, out_v, sems)


def _mlp_body(x_ref, w_ref, b_ref, o_ref):
    o_ref[...] = (
        jnp.dot(x_ref[...], w_ref[...], preferred_element_type=jnp.float32)
        + b_ref[...]
    )


def _mlp(x, w, b2):
    return pl.pallas_call(
        _mlp_body,
        out_shape=jax.ShapeDtypeStruct((B, CLS), jnp.float32),
    )(x, w, b2)


def kernel(tokenizedSeqArr, table, W, b):
    yT = table.T
    tailT = lax.slice(yT, (0, TAIL_V0), (EMB, VOCAB))
    table2 = _transpose_sc(yT, tailT)
    pooled = _pool_sc(tokenizedSeqArr, table2)
    return _mlp(pooled, W, b.reshape(1, CLS))


# speedup vs baseline: 1.1611x; 1.1611x over previous
import functools

import jax
import jax.numpy as jnp
from jax import lax
from jax.experimental import pallas as pl
from jax.experimental.pallas import tpu as pltpu
from jax.experimental.pallas import tpu_sc as plsc

VOCAB = 1000000
EMB = 64
B = 4096
L = 200
CLS = 128

NC = 2
NS = 16
NW = NC * NS
ROWS_PER_W = B // NW
NLANE = EMB // 16

NBLK = 1954
D = NBLK * 256
VROWS2 = D

NSLOT = 4
UNROLL = 4


def _tr_body(a_ref, b_ref, o_ref):
    o_ref[...] = jnp.concatenate(
        [a_ref[...].T, b_ref[...].T], axis=1)


def _transpose_tc(yT):
    return pl.pallas_call(
        _tr_body,
        grid=(NBLK,),
        in_specs=[
            pl.BlockSpec((EMB, 256), lambda i: (0, i)),
            pl.BlockSpec(
                (EMB, 256),
                lambda i: (0, jnp.minimum(i + NBLK, VOCAB // 256))),
        ],
        out_specs=pl.BlockSpec((256, 2 * EMB), lambda i: (i, 0)),
        out_shape=jax.ShapeDtypeStruct((VROWS2, 2 * EMB), jnp.float32),
    )(yT, yT)


def _pool_body(seq_hbm, table_hbm, out_hbm, idx_v, buf_v, out_v, sems):
    wid = lax.axis_index("s") * NC + lax.axis_index("c")
    base = wid * ROWS_PER_W
    pltpu.sync_copy(seq_hbm.at[pl.ds(base, ROWS_PER_W)], idx_v)

    GSTARTS = tuple(range(0, L - 16, 16)) + (L - 16,)

    def rowidx(iv):
        return jnp.where(iv < D, iv * 2, (iv - D) * 2 + 1)

    def issue(row, slot):
        for st in GSTARTS:
            iv = rowidx(idx_v[row, pl.ds(st, 16)])
            pltpu.async_copy(
                table_hbm.at[iv], buf_v.at[slot, pl.ds(st, 16)], sems[slot])

    def drain(row, slot):
        for st in GSTARTS:
            iv = rowidx(idx_v[row, pl.ds(st, 16)])
            pltpu.make_async_copy(
                table_hbm.at[iv], buf_v.at[slot, pl.ds(st, 16)],
                sems[slot]).wait()

    def compute(i, slot):
        def max_body(j, acc):
            new = list(acc)
            for jj in range(UNROLL):
                for c in range(NLANE):
                    new[c] = jnp.maximum(
                        new[c],
                        buf_v[slot, UNROLL * j + jj, pl.ds(c * 16, 16)])
            return tuple(new)

        init = tuple(jnp.full((16,), -jnp.inf, jnp.float32)
                     for _ in range(NLANE))
        acc = lax.fori_loop(0, L // UNROLL, max_body, init)
        for c in range(NLANE):
            out_v[i, pl.ds(c * 16, 16)] = acc[c]

    for s in range(NSLOT):
        issue(s, s)

    def group_body(g, carry):
        r0 = NSLOT * g
        for s in range(NSLOT):
            drain(r0 + s, s)
            compute(r0 + s, s)

            @pl.when(r0 + s + NSLOT < ROWS_PER_W)
            def _():
                issue(r0 + s + NSLOT, s)
        return carry

    lax.fori_loop(0, ROWS_PER_W // NSLOT, group_body, 0)
    pltpu.sync_copy(out_v, out_hbm.at[pl.ds(wid * ROWS_PER_W, ROWS_PER_W)])


@functools.partial(
    pl.kernel,
    out_type=jax.ShapeDtypeStruct((B, EMB), jnp.float32),
    mesh=plsc.VectorSubcoreMesh(core_axis_name="c", subcore_axis_name="s"),
    scratch_types=[
        pltpu.VMEM((ROWS_PER_W, L), jnp.int32),
        pltpu.VMEM((NSLOT, L, EMB), jnp.float32),
        pltpu.VMEM((ROWS_PER_W, EMB), jnp.float32),
    ] + [pltpu.SemaphoreType.DMA] * NSLOT,
    compiler_params=pltpu.CompilerParams(use_tc_tiling_on_sc=False),
)
def _pool_sc(seq_hbm, table_hbm, out_hbm, idx_v, buf_v, out_v, *sems):
    _pool_body(seq_hbm, table_hbm, out_hbm, idx_v, buf_v, out_v, sems)


def _mlp_body(x_ref, w_ref, b_ref, o_ref):
    o_ref[...] = (
        jnp.dot(x_ref[...], w_ref[...], preferred_element_type=jnp.float32)
        + b_ref[...]
    )


def _mlp(x, w, b2):
    return pl.pallas_call(
        _mlp_body,
        out_shape=jax.ShapeDtypeStruct((B, CLS), jnp.float32),
    )(x, w, b2)


def kernel(tokenizedSeqArr, table, W, b):
    yT = table.T
    t2 = _transpose_tc(yT)
    t2v = t2.reshape(2 * VROWS2, EMB)
    pooled = _pool_sc(tokenizedSeqArr, t2v)
    return _mlp(pooled, W, b.reshape(1, CLS))


# speedup vs baseline: 2.2044x; 1.8985x over previous
import functools

import jax
import jax.numpy as jnp
from jax import lax
from jax.experimental import pallas as pl
from jax.experimental.pallas import tpu as pltpu
from jax.experimental.pallas import tpu_sc as plsc

VOCAB = 1000000
EMB = 64
B = 4096
L = 200
CLS = 128

NC = 2
NS = 16
NW = NC * NS
ROWS_PER_W = B // NW
CHUNK = 100
CHUNKS_PER_ROW = L // CHUNK
NCHUNK = ROWS_PER_W * CHUNKS_PER_ROW
NLANE = EMB // 16


NSLOT = 4
UNROLL = 4


def _pool_body(seq_hbm, table_hbm, out_hbm, idx_v, buf_v, out_v, sems):
    wid = lax.axis_index("s") * NC + lax.axis_index("c")
    base = wid * NCHUNK
    pltpu.sync_copy(seq_hbm.at[pl.ds(base, NCHUNK)], idx_v)

    def issue(row, slot):
        for half in range(CHUNKS_PER_ROW):
            pltpu.async_copy(
                table_hbm.at[idx_v.at[CHUNKS_PER_ROW * row + half]],
                buf_v.at[slot, half], sems[slot])

    def drain(row, slot):
        for half in range(CHUNKS_PER_ROW):
            pltpu.make_async_copy(
                table_hbm.at[idx_v.at[CHUNKS_PER_ROW * row + half]],
                buf_v.at[slot, half], sems[slot]).wait()

    def compute(i, slot):
        def max_body(j, acc):
            new = list(acc)
            for jj in range(UNROLL):
                for c in range(NLANE):
                    m = new[c]
                    for half in range(CHUNKS_PER_ROW):
                        m = jnp.maximum(
                            m,
                            buf_v[slot, half, UNROLL * j + jj,
                                  pl.ds(c * 16, 16)])
                    new[c] = m
            return tuple(new)

        init = tuple(jnp.full((16,), -jnp.inf, jnp.float32)
                     for _ in range(NLANE))
        acc = lax.fori_loop(0, CHUNK // UNROLL, max_body, init)
        for c in range(NLANE):
            out_v[i, pl.ds(c * 16, 16)] = acc[c]

    for s in range(NSLOT):
        issue(s, s)

    def group_body(g, carry):
        r0 = NSLOT * g
        for s in range(NSLOT):
            drain(r0 + s, s)
            compute(r0 + s, s)

            @pl.when(r0 + s + NSLOT < ROWS_PER_W)
            def _():
                issue(r0 + s + NSLOT, s)
        return carry

    lax.fori_loop(0, ROWS_PER_W // NSLOT, group_body, 0)
    pltpu.sync_copy(out_v, out_hbm.at[pl.ds(wid * ROWS_PER_W, ROWS_PER_W)])


@functools.partial(
    pl.kernel,
    out_type=jax.ShapeDtypeStruct((B, EMB), jnp.float32),
    mesh=plsc.VectorSubcoreMesh(core_axis_name="c", subcore_axis_name="s"),
    scratch_types=[
        pltpu.VMEM((NCHUNK, CHUNK), jnp.int32),
        pltpu.VMEM((NSLOT, CHUNKS_PER_ROW, CHUNK, EMB), jnp.float32),
        pltpu.VMEM((ROWS_PER_W, EMB), jnp.float32),
    ] + [pltpu.SemaphoreType.DMA] * NSLOT,
    compiler_params=pltpu.CompilerParams(use_tc_tiling_on_sc=False),
)
def _pool_sc(seq_hbm, table_hbm, out_hbm, idx_v, buf_v, out_v, *sems):
    _pool_body(seq_hbm, table_hbm, out_hbm, idx_v, buf_v, out_v, sems)


def _mlp_body(x_ref, w_ref, b_ref, o_ref):
    o_ref[...] = (
        jnp.dot(x_ref[...], w_ref[...], preferred_element_type=jnp.float32)
        + b_ref[...]
    )


def _mlp(x, w, b2):
    return pl.pallas_call(
        _mlp_body,
        out_shape=jax.ShapeDtypeStruct((B, CLS), jnp.float32),
    )(x, w, b2)


def kernel(tokenizedSeqArr, table, W, b):
    seq2 = tokenizedSeqArr.reshape(B * CHUNKS_PER_ROW, CHUNK)
    pooled = _pool_sc(seq2, table)
    return _mlp(pooled, W, b.reshape(1, CLS))
